# Initial kernel scaffold; baseline (speedup 1.0000x reference)
import functools

import jax
import jax.numpy as jnp
from jax import lax
from jax.experimental import pallas as pl
from jax.experimental.pallas import tpu as pltpu
from jax.experimental.pallas import tpu_sc as plsc

L = 16
NSUB = 16
NCORE = 2
NW = NSUB * NCORE
CHUNK_ROWS = 16
CHUNK = CHUNK_ROWS * 128


def _pack_table_tc(an, a_exponent, np_pad):
    n = an.shape[0]
    an_p = jnp.concatenate([an, jnp.ones((np_pad - n,), jnp.float32)])
    an_2d = an_p.reshape(np_pad // 128, 128)
    ae = jnp.reshape(a_exponent.astype(jnp.float32), (1, 1))

    def body(an_ref, ae_ref, out_ref):
        a = jnp.maximum(an_ref[...], 1e-6)
        za = jnp.exp(jnp.log(a) * jnp.abs(ae_ref[0, 0]))
        hi = lax.bitcast_convert_type(a.astype(jnp.bfloat16), jnp.uint16)
        lo = lax.bitcast_convert_type(za.astype(jnp.bfloat16), jnp.uint16)
        w = (hi.astype(jnp.uint32) << 16) | lo.astype(jnp.uint32)
        out_ref[...] = lax.bitcast_convert_type(w, jnp.int32)

    out = pl.pallas_call(
        body,
        out_shape=jax.ShapeDtypeStruct(an_2d.shape, jnp.int32),
    )(an_2d, ae)
    return out.reshape(-1)


def _sc_edge_kernel(np_pad, nchunks, kiters):
    sl = np_pad // NSUB
    mesh = plsc.VectorSubcoreMesh(core_axis_name="c", subcore_axis_name="s")

    @functools.partial(
        pl.kernel,
        out_type=jax.ShapeDtypeStruct((NCORE, NSUB, sl), jnp.float32),
        mesh=mesh,
        scratch_types=[
            pltpu.VMEM((np_pad,), jnp.int32),
            pltpu.VMEM((8,), jnp.float32),
            pltpu.VMEM((CHUNK_ROWS, 128), jnp.int32),
            pltpu.VMEM((CHUNK_ROWS, 128), jnp.int32),
            pltpu.VMEM((CHUNK_ROWS, 128), jnp.float32),
            pltpu.VMEM((CHUNK_ROWS, 128), jnp.float32),
            pltpu.VMEM((CHUNK_ROWS, 128), jnp.float32),
            pltpu.VMEM((CHUNK_ROWS, 128), jnp.float32),
            pltpu.VMEM_SHARED((np_pad,), jnp.float32),
            pltpu.SemaphoreType.DMA,
        ],
    )
    def kern(table_h, idxi_h, idxj_h, d_h, sw_h, es_h, par_h, zer_h, out_h,
             table_v, par_v, ii_v, jj_v, d_v, sw_v, es_v, rep_v, acc_sh, sem):
        c = lax.axis_index("c")
        s = lax.axis_index("s")
        w = s * NCORE + c

        pltpu.sync_copy(table_h, table_v)
        pltpu.sync_copy(par_h, par_v)

        @pl.when(s == 0)
        def _():
            pltpu.sync_copy(zer_h, acc_sh)

        plsc.subcore_barrier()

        c0 = par_v[0]
        c1 = par_v[1]
        c2 = par_v[2]
        c3 = par_v[3]
        f0 = par_v[4]
        f1 = par_v[5]
        f2 = par_v[6]
        f3 = par_v[7]
        himask = jnp.int32(-65536)

        def chunk_body(k, carry):
            cid = k * NW + w

            @pl.when(cid < nchunks)
            def _():
                r0 = cid * CHUNK_ROWS
                cps = [
                    pltpu.async_copy(idxi_h.at[pl.ds(r0, CHUNK_ROWS)], ii_v, sem),
                    pltpu.async_copy(idxj_h.at[pl.ds(r0, CHUNK_ROWS)], jj_v, sem),
                    pltpu.async_copy(d_h.at[pl.ds(r0, CHUNK_ROWS)], d_v, sem),
                    pltpu.async_copy(sw_h.at[pl.ds(r0, CHUNK_ROWS)], sw_v, sem),
                    pltpu.async_copy(es_h.at[pl.ds(r0, CHUNK_ROWS)], es_v, sem),
                ]
                for cp in cps:
                    cp.wait()

                def row_body(r, rcarry):
                    for cb in range(128 // L):
                        col = pl.ds(cb * L, L)
                        ii = ii_v[r, col]
                        jj = jj_v[r, col]
                        wi = plsc.load_gather(table_v, [ii])
                        wj = plsc.load_gather(table_v, [jj])
                        ani = plsc.bitcast(wi & himask, jnp.float32)
                        zai = plsc.bitcast(wi << 16, jnp.float32)
                        anj = plsc.bitcast(wj & himask, jnp.float32)
                        zaj = plsc.bitcast(wj << 16, jnp.float32)
                        dd = jnp.maximum(d_v[r, col], 1e-6)
                        t = dd * jnp.maximum(zai + zaj, 1e-10)
                        p = (c0 * jnp.exp(f0 * t) + c1 * jnp.exp(f1 * t)
                             + c2 * jnp.exp(f2 * t) + c3 * jnp.exp(f3 * t))
                        es = jnp.maximum(es_v[r, col], 0.0)
                        sw = jnp.maximum(sw_v[r, col], 0.0)
                        rep_v[r, col] = ((ani * anj) / dd * p + es) * sw
                    return rcarry

                lax.fori_loop(0, CHUNK_ROWS, row_body, 0)
                pltpu.sync_copy(rep_v, acc_sh.at[ii_v], add=True)

            return carry

        lax.fori_loop(0, kiters, chunk_body, 0)
        plsc.subcore_barrier()
        pltpu.sync_copy(acc_sh.at[pl.ds(s * sl, sl)], out_h.at[c, s])

    return kern


def kernel(atomic_numbers, distances, switch_off, eshift, idx_i, idx_j,
           atom_mask, batch_mask, batch_segments, batch_size,
           a_coefficient, a_exponent, phi_coefficients, phi_exponents):
    n = atomic_numbers.shape[0]
    e = distances.shape[0]
    np_pad = ((n + NSUB * 8 - 1) // (NSUB * 8)) * (NSUB * 8)
    np_pad = ((np_pad + 127) // 128) * 128
    erows = e // 128
    nchunks = erows // CHUNK_ROWS
    kiters = (nchunks + NW - 1) // NW

    table = _pack_table_tc(atomic_numbers, a_exponent, np_pad)

    a_abs = jnp.maximum(jnp.abs(a_coefficient), 1e-10)
    rc = jnp.abs(phi_coefficients)
    coeffs = rc / jnp.maximum(jnp.sum(rc), 1e-10)
    fexp = -phi_exponents / a_abs
    params = jnp.concatenate([coeffs, fexp]).astype(jnp.float32)

    idxi2 = idx_i.reshape(erows, 128)
    idxj2 = idx_j.reshape(erows, 128)
    d2 = distances.reshape(erows, 128)
    sw2 = switch_off.reshape(erows, 128)
    es2 = eshift.reshape(erows, 128)
    zeros = jnp.zeros((np_pad,), jnp.float32)

    kern = _sc_edge_kernel(np_pad, nchunks, kiters)
    parts = kern(table, idxi2, idxj2, d2, sw2, es2, params, zeros)

    erep = parts.reshape(NCORE, np_pad).sum(axis=0)[:n]
    erep = erep * atom_mask
    erep = jnp.nan_to_num(erep, nan=0.0, posinf=0.0, neginf=0.0)
    return erep[..., None, None, None]


# speedup vs baseline: 462.4308x; 462.4308x over previous
import functools

import jax
import jax.numpy as jnp
from jax import lax
from jax.experimental import pallas as pl
from jax.experimental.pallas import tpu as pltpu
from jax.experimental.pallas import tpu_sc as plsc

L = 16
NSUB = 16
NCORE = 2
NW = NSUB * NCORE
CHUNK_ROWS = 16
CHUNK = CHUNK_ROWS * 128


def _pack_table_tc(an, a_exponent, np_pad):
    n = an.shape[0]
    an_p = jnp.concatenate([an, jnp.ones((np_pad - n,), jnp.float32)])
    an_2d = an_p.reshape(np_pad // 128, 128)
    ae = jnp.reshape(a_exponent.astype(jnp.float32), (1, 1))

    def body(an_ref, ae_ref, out_ref):
        a = jnp.maximum(an_ref[...], 1e-6)
        za = jnp.exp(jnp.log(a) * jnp.abs(ae_ref[0, 0]))
        hi = lax.bitcast_convert_type(a.astype(jnp.bfloat16), jnp.uint16)
        lo = lax.bitcast_convert_type(za.astype(jnp.bfloat16), jnp.uint16)
        w = (hi.astype(jnp.uint32) << 16) | lo.astype(jnp.uint32)
        out_ref[...] = lax.bitcast_convert_type(w, jnp.int32)

    out = pl.pallas_call(
        body,
        out_shape=jax.ShapeDtypeStruct(an_2d.shape, jnp.int32),
    )(an_2d, ae)
    return out.reshape(-1)


def _sc_edge_kernel(np_pad, nchunks, kiters):
    sl = np_pad // NSUB
    mesh = plsc.VectorSubcoreMesh(core_axis_name="c", subcore_axis_name="s")

    @functools.partial(
        pl.kernel,
        out_type=jax.ShapeDtypeStruct((NCORE, NSUB, sl), jnp.float32),
        mesh=mesh,
        compiler_params=pltpu.CompilerParams(needs_layout_passes=False),
        scratch_types=[
            pltpu.VMEM((np_pad,), jnp.int32),
            pltpu.VMEM((16,), jnp.float32),
            pltpu.VMEM((CHUNK,), jnp.int32),
            pltpu.VMEM((CHUNK,), jnp.int32),
            pltpu.VMEM((CHUNK,), jnp.float32),
            pltpu.VMEM((CHUNK,), jnp.float32),
            pltpu.VMEM((CHUNK,), jnp.float32),
            pltpu.VMEM((CHUNK,), jnp.float32),
            pltpu.VMEM_SHARED((np_pad,), jnp.float32),
            pltpu.SemaphoreType.DMA,
        ],
    )
    def kern(table_h, idxi_h, idxj_h, d_h, sw_h, es_h, par_h, zer_h, out_h,
             table_v, par_v, ii_v, jj_v, d_v, sw_v, es_v, rep_v, acc_sh, sem):
        c = lax.axis_index("c")
        s = lax.axis_index("s")
        w = s * NCORE + c

        pltpu.sync_copy(table_h, table_v)
        pltpu.sync_copy(par_h, par_v)

        @pl.when(s == 0)
        def _():
            pltpu.sync_copy(zer_h, acc_sh)

        plsc.subcore_barrier()

        pv = par_v[...]
        c0 = pv[0]
        c1 = pv[1]
        c2 = pv[2]
        c3 = pv[3]
        f0 = pv[4]
        f1 = pv[5]
        f2 = pv[6]
        f3 = pv[7]
        himask = jnp.int32(-65536)

        def chunk_body(k, carry):
            cid = k * NW + w

            @pl.when(cid < nchunks)
            def _():
                e0 = cid * CHUNK
                cps = [
                    pltpu.async_copy(idxi_h.at[pl.ds(e0, CHUNK)], ii_v, sem),
                    pltpu.async_copy(idxj_h.at[pl.ds(e0, CHUNK)], jj_v, sem),
                    pltpu.async_copy(d_h.at[pl.ds(e0, CHUNK)], d_v, sem),
                    pltpu.async_copy(sw_h.at[pl.ds(e0, CHUNK)], sw_v, sem),
                    pltpu.async_copy(es_h.at[pl.ds(e0, CHUNK)], es_v, sem),
                ]
                for cp in cps:
                    cp.wait()

                def vec_body(v, rcarry):
                    col = pl.ds(v * L, L)
                    ii = ii_v[col]
                    jj = jj_v[col]
                    wi = plsc.load_gather(table_v, [ii])
                    wj = plsc.load_gather(table_v, [jj])
                    ani = plsc.bitcast(wi & himask, jnp.float32)
                    zai = plsc.bitcast(wi << 16, jnp.float32)
                    anj = plsc.bitcast(wj & himask, jnp.float32)
                    zaj = plsc.bitcast(wj << 16, jnp.float32)
                    dd = jnp.maximum(d_v[col], 1e-6)
                    t = dd * jnp.maximum(zai + zaj, 1e-10)
                    p = (c0 * jnp.exp(f0 * t) + c1 * jnp.exp(f1 * t)
                         + c2 * jnp.exp(f2 * t) + c3 * jnp.exp(f3 * t))
                    es = jnp.maximum(es_v[col], 0.0)
                    sw = jnp.maximum(sw_v[col], 0.0)
                    rep_v[col] = ((ani * anj) / dd * p + es) * sw
                    return rcarry

                lax.fori_loop(0, CHUNK // L, vec_body, 0)
                pltpu.sync_copy(rep_v, acc_sh.at[ii_v], add=True)

            return carry

        lax.fori_loop(0, kiters, chunk_body, 0)
        plsc.subcore_barrier()
        pltpu.sync_copy(acc_sh.at[pl.ds(s * sl, sl)], out_h.at[c, s])

    return kern


def kernel(atomic_numbers, distances, switch_off, eshift, idx_i, idx_j,
           atom_mask, batch_mask, batch_segments, batch_size,
           a_coefficient, a_exponent, phi_coefficients, phi_exponents):
    n = atomic_numbers.shape[0]
    e = distances.shape[0]
    np_pad = ((n + NSUB * 128 - 1) // (NSUB * 128)) * (NSUB * 128)
    nchunks = e // CHUNK
    kiters = (nchunks + NW - 1) // NW

    table = _pack_table_tc(atomic_numbers, a_exponent, np_pad)

    a_abs = jnp.maximum(jnp.abs(a_coefficient), 1e-10)
    rc = jnp.abs(phi_coefficients)
    coeffs = rc / jnp.maximum(jnp.sum(rc), 1e-10)
    fexp = -phi_exponents / a_abs
    params = jnp.concatenate(
        [coeffs, fexp, jnp.zeros((8,), jnp.float32)]).astype(jnp.float32)

    zeros = jnp.zeros((np_pad,), jnp.float32)

    kern = _sc_edge_kernel(np_pad, nchunks, kiters)
    parts = kern(table, idx_i, idx_j, distances, switch_off, eshift,
                 params, zeros)

    erep = parts.reshape(NCORE, np_pad).sum(axis=0)[:n]
    erep = erep * atom_mask
    erep = jnp.nan_to_num(erep, nan=0.0, posinf=0.0, neginf=0.0)
    return erep[..., None, None, None]
